# Initial kernel scaffold; baseline (speedup 1.0000x reference)
import functools

import jax
import jax.numpy as jnp
from jax import lax
from jax.experimental import pallas as pl
from jax.experimental.pallas import tpu as pltpu
from jax.experimental.pallas import tpu_sc as plsc

N_GRAPHS = 512
ROWS_PER_BLOCK = 1024
N_WORKERS = 16
BIN_PAD = 528


def _tc_dot_body(x_ref, w_ref, o_ref):
    s = jnp.sum(x_ref[...] * w_ref[...], axis=1)
    o_ref[...] = s.reshape(1, 1, ROWS_PER_BLOCK)


def _sc_segment_sum(y_w, b_w, n_chunks):
    mesh = plsc.VectorSubcoreMesh(core_axis_name="c", subcore_axis_name="s")

    @functools.partial(
        pl.kernel,
        out_type=jax.ShapeDtypeStruct((N_GRAPHS,), jnp.float32),
        mesh=mesh,
        scratch_types=[
            pltpu.VMEM((n_chunks, 128), jnp.float32),
            pltpu.VMEM((n_chunks, 128), jnp.int32),
            pltpu.VMEM((BIN_PAD,), jnp.float32),
            pltpu.VMEM_SHARED((BIN_PAD,), jnp.float32),
        ],
    )
    def seg_sum(y_hbm, b_hbm, out_hbm, val_v, idx_v, zero_v, bins_sh):
        c = lax.axis_index("c")
        s = lax.axis_index("s")

        @pl.when(c == 0)
        def _core0():
            @pl.when(s == 0)
            def _zero_bins():
                for j in range(BIN_PAD // 16):
                    zero_v[pl.ds(16 * j, 16)] = jnp.zeros((16,), jnp.float32)
                pltpu.sync_copy(zero_v, bins_sh)

            plsc.subcore_barrier()

            pltpu.sync_copy(y_hbm.at[s], val_v)
            pltpu.sync_copy(b_hbm.at[s], idx_v)
            for j in range(n_chunks):
                pltpu.sync_copy(val_v.at[j], bins_sh.at[idx_v.at[j]], add=True)

            plsc.subcore_barrier()

            @pl.when(s == 0)
            def _write_out():
                pltpu.sync_copy(bins_sh.at[pl.ds(0, N_GRAPHS)], out_hbm)

    return seg_sum(y_w, b_w)


def kernel(node_features, batch, W):
    n, d = node_features.shape
    n_blocks = -(-n // ROWS_PER_BLOCK)
    n_pad = n_blocks * ROWS_PER_BLOCK

    y3 = pl.pallas_call(
        _tc_dot_body,
        grid=(n_blocks,),
        in_specs=[
            pl.BlockSpec((ROWS_PER_BLOCK, d), lambda i: (i, 0)),
            pl.BlockSpec((1, d), lambda i: (0, 0)),
        ],
        out_specs=pl.BlockSpec((1, 1, ROWS_PER_BLOCK), lambda i: (i, 0, 0)),
        out_shape=jax.ShapeDtypeStruct((n_blocks, 1, ROWS_PER_BLOCK), jnp.float32),
    )(node_features, W.reshape(1, d))
    y = y3.reshape(n_pad)

    b_pad = jnp.pad(batch.astype(jnp.int32), (0, n_pad - n),
                    constant_values=N_GRAPHS)

    per_w = n_pad // N_WORKERS
    n_chunks = per_w // 128
    y_w = y.reshape(N_WORKERS, n_chunks, 128)
    b_w = b_pad.reshape(N_WORKERS, n_chunks, 128)

    out = _sc_segment_sum(y_w, b_w, n_chunks)
    return out.reshape(N_GRAPHS, 1)


# speedup vs baseline: 3.2346x; 3.2346x over previous
import functools

import jax
import jax.numpy as jnp
from jax import lax
from jax.experimental import pallas as pl
from jax.experimental.pallas import tpu as pltpu
from jax.experimental.pallas import tpu_sc as plsc

N_GRAPHS = 512
ROWS_PER_BLOCK = 1024
N_WORKERS = 16
BIN_PAD = 528


def _tc_dot_body(x_ref, w_ref, o_ref):
    s = jnp.sum(x_ref[...] * w_ref[...], axis=1)
    o_ref[...] = s.reshape(1, 1, ROWS_PER_BLOCK)


def _sc_segment_sum(y_w, b_w, n_chunks):
    mesh = plsc.VectorSubcoreMesh(core_axis_name="c", subcore_axis_name="s",
                                  num_cores=2, num_subcores=16)

    @functools.partial(
        pl.kernel,
        out_type=jax.ShapeDtypeStruct((N_GRAPHS,), jnp.float32),
        mesh=mesh,
        scratch_types=[
            pltpu.VMEM((n_chunks, 128), jnp.float32),
            pltpu.VMEM((n_chunks, 128), jnp.int32),
            pltpu.VMEM((BIN_PAD,), jnp.float32),
            pltpu.VMEM_SHARED((BIN_PAD,), jnp.float32),
        ],
    )
    def seg_sum(y_hbm, b_hbm, out_hbm, val_v, idx_v, zero_v, bins_sh):
        c = lax.axis_index("c")
        s = lax.axis_index("s")

        @pl.when(c == 0)
        def _core0():
            @pl.when(s == 0)
            def _zero_bins():
                for j in range(BIN_PAD // 16):
                    zero_v[pl.ds(16 * j, 16)] = jnp.zeros((16,), jnp.float32)
                pltpu.sync_copy(zero_v, bins_sh)

            plsc.subcore_barrier()

            pltpu.sync_copy(y_hbm.at[s], val_v)
            pltpu.sync_copy(b_hbm.at[s], idx_v)
            for j in range(n_chunks):
                pltpu.sync_copy(val_v.at[j], bins_sh.at[idx_v.at[j]], add=True)

            plsc.subcore_barrier()

            @pl.when(s == 0)
            def _write_out():
                pltpu.sync_copy(bins_sh.at[pl.ds(0, N_GRAPHS)], out_hbm)

    return seg_sum(y_w, b_w)


def kernel(node_features, batch, W):
    n, d = node_features.shape
    n_blocks = -(-n // ROWS_PER_BLOCK)
    n_pad = n_blocks * ROWS_PER_BLOCK

    y3 = pl.pallas_call(
        _tc_dot_body,
        grid=(n_blocks,),
        in_specs=[
            pl.BlockSpec((ROWS_PER_BLOCK, d), lambda i: (i, 0)),
            pl.BlockSpec((1, d), lambda i: (0, 0)),
        ],
        out_specs=pl.BlockSpec((1, 1, ROWS_PER_BLOCK), lambda i: (i, 0, 0)),
        out_shape=jax.ShapeDtypeStruct((n_blocks, 1, ROWS_PER_BLOCK), jnp.float32),
    )(node_features, W.reshape(1, d))
    y = y3.reshape(n_pad)

    b_pad = jnp.pad(batch.astype(jnp.int32), (0, n_pad - n),
                    constant_values=N_GRAPHS)

    per_w = n_pad // N_WORKERS
    n_chunks = per_w // 128
    y_w = y.reshape(N_WORKERS, n_chunks, 128)
    b_w = b_pad.reshape(N_WORKERS, n_chunks, 128)

    out = _sc_segment_sum(y_w, b_w, n_chunks)
    return out.reshape(N_GRAPHS, 1)


# speedup vs baseline: 3.5581x; 1.1000x over previous
import functools

import jax
import jax.numpy as jnp
from jax import lax
from jax.experimental import pallas as pl
from jax.experimental.pallas import tpu as pltpu
from jax.experimental.pallas import tpu_sc as plsc

N_GRAPHS = 512
ROWS_PER_BLOCK = 2048
N_WORKERS = 16
BIN_PAD = 528


def _tc_dot_body(x_ref, w_ref, o_ref):
    s = jax.lax.dot_general(x_ref[...], w_ref[...],
                            (((1,), (0,)), ((), ())),
                            preferred_element_type=jnp.float32)
    o_ref[...] = s.reshape(1, ROWS_PER_BLOCK, 1)


def _sc_segment_sum(y_w, b_w, n_chunks):
    mesh = plsc.VectorSubcoreMesh(core_axis_name="c", subcore_axis_name="s",
                                  num_cores=2, num_subcores=16)

    @functools.partial(
        pl.kernel,
        out_type=jax.ShapeDtypeStruct((N_GRAPHS,), jnp.float32),
        mesh=mesh,
        scratch_types=[
            pltpu.VMEM((n_chunks, 128), jnp.float32),
            pltpu.VMEM((n_chunks, 128), jnp.int32),
            pltpu.VMEM((BIN_PAD,), jnp.float32),
            pltpu.VMEM_SHARED((BIN_PAD,), jnp.float32),
        ],
    )
    def seg_sum(y_hbm, b_hbm, out_hbm, val_v, idx_v, zero_v, bins_sh):
        c = lax.axis_index("c")
        s = lax.axis_index("s")

        @pl.when(c == 0)
        def _core0():
            @pl.when(s == 0)
            def _zero_bins():
                for j in range(BIN_PAD // 16):
                    zero_v[pl.ds(16 * j, 16)] = jnp.zeros((16,), jnp.float32)
                pltpu.sync_copy(zero_v, bins_sh)

            plsc.subcore_barrier()

            pltpu.sync_copy(y_hbm.at[s], val_v)
            pltpu.sync_copy(b_hbm.at[s], idx_v)
            for j in range(n_chunks):
                pltpu.sync_copy(val_v.at[j], bins_sh.at[idx_v.at[j]], add=True)

            plsc.subcore_barrier()

            @pl.when(s == 0)
            def _write_out():
                pltpu.sync_copy(bins_sh.at[pl.ds(0, N_GRAPHS)], out_hbm)

    return seg_sum(y_w, b_w)


def kernel(node_features, batch, W):
    n, d = node_features.shape
    n_blocks = -(-n // ROWS_PER_BLOCK)
    n_pad = n_blocks * ROWS_PER_BLOCK

    y3 = pl.pallas_call(
        _tc_dot_body,
        grid=(n_blocks,),
        in_specs=[
            pl.BlockSpec((ROWS_PER_BLOCK, d), lambda i: (i, 0)),
            pl.BlockSpec((d, 1), lambda i: (0, 0)),
        ],
        out_specs=pl.BlockSpec((1, ROWS_PER_BLOCK, 1), lambda i: (i, 0, 0)),
        out_shape=jax.ShapeDtypeStruct((n_blocks, ROWS_PER_BLOCK, 1), jnp.float32),
    )(node_features, W)
    y = y3.reshape(n_pad)

    b_pad = jnp.pad(batch.astype(jnp.int32), (0, n_pad - n),
                    constant_values=N_GRAPHS)

    per_w = n_pad // N_WORKERS
    n_chunks = per_w // 128
    y_w = y.reshape(N_WORKERS, n_chunks, 128)
    b_w = b_pad.reshape(N_WORKERS, n_chunks, 128)

    out = _sc_segment_sum(y_w, b_w, n_chunks)
    return out.reshape(N_GRAPHS, 1)


# speedup vs baseline: 7.1902x; 2.0208x over previous
import functools

import jax
import jax.numpy as jnp
from jax import lax
from jax.experimental import pallas as pl
from jax.experimental.pallas import tpu as pltpu
from jax.experimental.pallas import tpu_sc as plsc

N_GRAPHS = 512
ROWS_PER_BLOCK = 7168
SUB_ROWS = ROWS_PER_BLOCK // 8
N_WORKERS = 16
BIN_PAD = 528


def _tc_dot_body(x_ref, w_ref, o_ref):
    w = w_ref[...]
    for j in range(8):
        s = jax.lax.dot_general(w, x_ref[pl.ds(j * SUB_ROWS, SUB_ROWS), :],
                                (((1,), (1,)), ((), ())),
                                preferred_element_type=jnp.float32)
        o_ref[0, j, :] = s[0]


def _sc_segment_sum(y_w, b_w, n_chunks):
    mesh = plsc.VectorSubcoreMesh(core_axis_name="c", subcore_axis_name="s",
                                  num_cores=2, num_subcores=16)

    @functools.partial(
        pl.kernel,
        out_type=jax.ShapeDtypeStruct((N_GRAPHS,), jnp.float32),
        mesh=mesh,
        scratch_types=[
            pltpu.VMEM((n_chunks, 128), jnp.float32),
            pltpu.VMEM((n_chunks, 128), jnp.int32),
            pltpu.VMEM((BIN_PAD,), jnp.float32),
            pltpu.VMEM_SHARED((BIN_PAD,), jnp.float32),
        ],
    )
    def seg_sum(y_hbm, b_hbm, out_hbm, val_v, idx_v, zero_v, bins_sh):
        c = lax.axis_index("c")
        s = lax.axis_index("s")

        @pl.when(c == 0)
        def _core0():
            @pl.when(s == 0)
            def _zero_bins():
                for j in range(BIN_PAD // 16):
                    zero_v[pl.ds(16 * j, 16)] = jnp.zeros((16,), jnp.float32)
                pltpu.sync_copy(zero_v, bins_sh)

            plsc.subcore_barrier()

            pltpu.sync_copy(y_hbm.at[s], val_v)
            pltpu.sync_copy(b_hbm.at[s], idx_v)
            for j in range(n_chunks):
                pltpu.sync_copy(val_v.at[j], bins_sh.at[idx_v.at[j]], add=True)

            plsc.subcore_barrier()

            @pl.when(s == 0)
            def _write_out():
                pltpu.sync_copy(bins_sh.at[pl.ds(0, N_GRAPHS)], out_hbm)

    return seg_sum(y_w, b_w)


def kernel(node_features, batch, W):
    n, d = node_features.shape
    n_blocks = -(-n // ROWS_PER_BLOCK)
    n_pad = n_blocks * ROWS_PER_BLOCK

    y3 = pl.pallas_call(
        _tc_dot_body,
        grid=(n_blocks,),
        in_specs=[
            pl.BlockSpec((ROWS_PER_BLOCK, d), lambda i: (i, 0)),
            pl.BlockSpec((1, d), lambda i: (0, 0)),
        ],
        out_specs=pl.BlockSpec((1, 8, SUB_ROWS), lambda i: (i, 0, 0)),
        out_shape=jax.ShapeDtypeStruct((n_blocks, 8, SUB_ROWS), jnp.float32),
    )(node_features, W.reshape(1, d))
    y = y3.reshape(n_pad)

    b_pad = jnp.pad(batch.astype(jnp.int32), (0, n_pad - n),
                    constant_values=N_GRAPHS)

    per_w = n_pad // N_WORKERS
    n_chunks = per_w // 128
    y_w = y.reshape(N_WORKERS, n_chunks, 128)
    b_w = b_pad.reshape(N_WORKERS, n_chunks, 128)

    out = _sc_segment_sum(y_w, b_w, n_chunks)
    return out.reshape(N_GRAPHS, 1)
